# baseline (speedup 1.0000x reference)
import functools

import jax
import jax.numpy as jnp
from jax import lax
from jax.experimental import pallas as pl
from jax.experimental.pallas import tpu as pltpu
from jax.experimental.pallas import tpu_sc as plsc

_B, _L, _D = 4096, 200, 32
_CH = 128


def _make_kernel():
    mesh = plsc.VectorSubcoreMesh(core_axis_name="c", subcore_axis_name="s")
    nc, ns = mesh.num_cores, mesh.num_subcores
    nw = nc * ns
    assert _B // _CH == nw

    @functools.partial(
        pl.kernel,
        out_type=jax.ShapeDtypeStruct((_L, _D // 8, nw, 8, _CH), jnp.float32),
        mesh=mesh,
        compiler_params=pltpu.CompilerParams(use_tc_tiling_on_sc=False,
                                             needs_layout_passes=False),
        scratch_types=[
            pltpu.VMEM((_L, _CH), jnp.int32),
            pltpu.VMEM((_L, _D), jnp.float32),
            pltpu.VMEM((_CH, _D), jnp.float32),
            pltpu.VMEM((_CH, _D), jnp.float32),
            pltpu.VMEM((_D // 8, 8, _CH), jnp.float32),
            pltpu.VMEM((_D // 8, 8, _CH), jnp.float32),
            pltpu.SemaphoreType.DMA,
            pltpu.SemaphoreType.DMA,
            pltpu.SemaphoreType.DMA,
            pltpu.SemaphoreType.DMA,
        ],
    )
    def emb_kernel(tok_hbm, xt_hbm, pos_hbm, out_hbm,
                   idx_v, pos_v, gbuf0, gbuf1, tbuf0, tbuf1,
                   gsem0, gsem1, ssem0, ssem1):
        wid = lax.axis_index("s") * nc + lax.axis_index("c")

        pltpu.sync_copy(xt_hbm.at[:, pl.ds(wid * _CH, _CH)], idx_v)
        pltpu.sync_copy(pos_hbm, pos_v)

        pltpu.async_copy(tok_hbm.at[idx_v.at[0]], gbuf0, gsem0)

        lane = lax.iota(jnp.int32, 16)
        rows = [lane + (16 * j) for j in range(8)]

        def step(g, gbuf_b, gsem_b, ssem_b, tbuf_b, gbuf_n, gsem_n, ssem_n,
                 tbuf_n):
            @pl.when(g >= 1)
            def _():
                pltpu.make_async_copy(
                    tbuf_n, out_hbm.at[g - 1, :, wid], ssem_n).wait()

            @pl.when(g + 1 < _L)
            def _():
                pltpu.async_copy(tok_hbm.at[idx_v.at[g + 1]], gbuf_n, gsem_n)

            pltpu.make_async_copy(tok_hbm.at[idx_v.at[g]], gbuf_b, gsem_b).wait()

            grow = jnp.full((16,), g, jnp.int32)
            for d in range(_D):
                col = jnp.full((16,), d, jnp.int32)
                padd = plsc.load_gather(pos_v, [grow, col])
                for j in range(8):
                    v = plsc.load_gather(gbuf_b, [rows[j], col])
                    tbuf_b[d // 8, d % 8, pl.ds(16 * j, 16)] = v + padd

            pltpu.async_copy(tbuf_b, out_hbm.at[g, :, wid], ssem_b)

        def outer(i, carry):
            g = i * 2
            step(g, gbuf0, gsem0, ssem0, tbuf0, gbuf1, gsem1, ssem1, tbuf1)
            step(g + 1, gbuf1, gsem1, ssem1, tbuf1, gbuf0, gsem0, ssem0, tbuf0)
            return carry

        lax.fori_loop(0, _L // 2, outer, 0)

        pltpu.make_async_copy(tbuf1, out_hbm.at[_L - 1, :, wid], ssem1).wait()

    return emb_kernel, nw


def kernel(x, token_table, pos_table):
    emb, nw = _make_kernel()
    xt = jnp.transpose(x.astype(jnp.int32))
    out5 = emb(token_table, xt, pos_table)
    out = jnp.transpose(out5, (2, 4, 0, 1, 3))
    return out.reshape(_B, _L, _D)


# speedup vs baseline: 1.2351x; 1.2351x over previous
import functools

import jax
import jax.numpy as jnp
from jax import lax
from jax.experimental import pallas as pl
from jax.experimental.pallas import tpu as pltpu
from jax.experimental.pallas import tpu_sc as plsc

_B, _L, _D = 4096, 200, 32
_CH = 128


def _make_kernel():
    mesh = plsc.VectorSubcoreMesh(core_axis_name="c", subcore_axis_name="s")
    nc, ns = mesh.num_cores, mesh.num_subcores
    nw = nc * ns
    assert _B // _CH == nw

    @functools.partial(
        pl.kernel,
        out_type=jax.ShapeDtypeStruct((_L, _D // 8, nw, 8, _CH), jnp.float32),
        mesh=mesh,
        compiler_params=pltpu.CompilerParams(use_tc_tiling_on_sc=False,
                                             needs_layout_passes=False),
        scratch_types=[
            pltpu.VMEM((_L, _CH), jnp.int32),
            pltpu.VMEM((_L, _D), jnp.float32),
            pltpu.VMEM((_CH, _D), jnp.float32),
            pltpu.VMEM((_CH, _D), jnp.float32),
            pltpu.VMEM((_D // 8, 8, _CH), jnp.float32),
            pltpu.VMEM((_D // 8, 8, _CH), jnp.float32),
            pltpu.SemaphoreType.DMA,
            pltpu.SemaphoreType.DMA,
            pltpu.SemaphoreType.DMA,
            pltpu.SemaphoreType.DMA,
        ],
    )
    def emb_kernel(tok_hbm, xt_hbm, pos_hbm, out_hbm,
                   idx_v, pos_v, gbuf0, gbuf1, tbuf0, tbuf1,
                   gsem0, gsem1, ssem0, ssem1):
        wid = lax.axis_index("s") * nc + lax.axis_index("c")

        pltpu.sync_copy(xt_hbm.at[:, pl.ds(wid * _CH, _CH)], idx_v)
        pltpu.sync_copy(pos_hbm, pos_v)

        pltpu.async_copy(tok_hbm.at[idx_v.at[0]], gbuf0, gsem0)

        lane = lax.iota(jnp.int32, 16)
        rows = [lane + (16 * j) for j in range(8)]

        def step(g, gbuf_b, gsem_b, ssem_b, tbuf_b, gbuf_n, gsem_n, ssem_n,
                 tbuf_n):
            @pl.when(g >= 1)
            def _():
                pltpu.make_async_copy(
                    tbuf_n, out_hbm.at[g - 1, :, wid], ssem_n).wait()

            @pl.when(g + 1 < _L)
            def _():
                pltpu.async_copy(tok_hbm.at[idx_v.at[g + 1]], gbuf_n, gsem_n)

            pltpu.make_async_copy(tok_hbm.at[idx_v.at[g]], gbuf_b, gsem_b).wait()

            grow = jnp.full((16,), g, jnp.int32)
            for d in range(_D):
                col = jnp.full((16,), d, jnp.int32)
                padd = plsc.load_gather(pos_v, [grow, col])
                vs = [plsc.load_gather(gbuf_b, [rows[j], col])
                      for j in range(8)]
                for j in range(8):
                    tbuf_b[d // 8, d % 8, pl.ds(16 * j, 16)] = vs[j] + padd

            pltpu.async_copy(tbuf_b, out_hbm.at[g, :, wid], ssem_b)

        def outer(i, carry):
            g = i * 2
            step(g, gbuf0, gsem0, ssem0, tbuf0, gbuf1, gsem1, ssem1, tbuf1)
            step(g + 1, gbuf1, gsem1, ssem1, tbuf1, gbuf0, gsem0, ssem0, tbuf0)
            return carry

        lax.fori_loop(0, _L // 2, outer, 0)

        pltpu.make_async_copy(tbuf1, out_hbm.at[_L - 1, :, wid], ssem1).wait()

    return emb_kernel, nw


def kernel(x, token_table, pos_table):
    emb, nw = _make_kernel()
    xt = jnp.transpose(x.astype(jnp.int32))
    out5 = emb(token_table, xt, pos_table)
    out = jnp.transpose(out5, (2, 4, 0, 1, 3))
    return out.reshape(_B, _L, _D)


# speedup vs baseline: 1.4007x; 1.1341x over previous
import functools

import jax
import jax.numpy as jnp
from jax import lax
from jax.experimental import pallas as pl
from jax.experimental.pallas import tpu as pltpu
from jax.experimental.pallas import tpu_sc as plsc

_B, _L, _D = 4096, 200, 32
_CH = 128


def _make_kernel():
    mesh = plsc.VectorSubcoreMesh(core_axis_name="c", subcore_axis_name="s")
    nc, ns = mesh.num_cores, mesh.num_subcores
    nw = nc * ns
    assert _B // _CH == nw

    @functools.partial(
        pl.kernel,
        out_type=jax.ShapeDtypeStruct((_L, _D // 8, nw, 8, _CH), jnp.float32),
        mesh=mesh,
        compiler_params=pltpu.CompilerParams(use_tc_tiling_on_sc=False,
                                             needs_layout_passes=False),
        scratch_types=[
            pltpu.VMEM((_L, _CH), jnp.int32),
            pltpu.VMEM((_L, _D), jnp.float32),
            pltpu.VMEM((_CH, _D), jnp.float32),
            pltpu.VMEM((_CH, _D), jnp.float32),
            pltpu.VMEM((_D // 8, 8, _CH), jnp.float32),
            pltpu.VMEM((_D // 8, 8, _CH), jnp.float32),
            pltpu.SemaphoreType.DMA,
            pltpu.SemaphoreType.DMA,
            pltpu.SemaphoreType.DMA,
            pltpu.SemaphoreType.DMA,
        ],
    )
    def emb_kernel(tok_hbm, xt_hbm, pos_hbm, out_hbm,
                   idx_v, pos_v, gbuf0, gbuf1, tbuf0, tbuf1,
                   gsem0, gsem1, ssem0, ssem1):
        wid = lax.axis_index("s") * nc + lax.axis_index("c")

        pltpu.sync_copy(xt_hbm.at[:, pl.ds(wid * _CH, _CH)], idx_v)
        pltpu.sync_copy(pos_hbm, pos_v)

        pltpu.async_copy(tok_hbm.at[idx_v.at[0]], gbuf0, gsem0)

        lane = lax.iota(jnp.int32, 16)
        rows = [lane + (16 * j) for j in range(8)]

        def step(g, gbuf_b, gsem_b, ssem_b, tbuf_b, gbuf_n, gsem_n, ssem_n,
                 tbuf_n):
            @pl.when(g >= 1)
            def _():
                pltpu.make_async_copy(
                    tbuf_n, out_hbm.at[g - 1, :, wid], ssem_n).wait()

            @pl.when(g + 1 < _L)
            def _():
                pltpu.async_copy(tok_hbm.at[idx_v.at[g + 1]], gbuf_n, gsem_n)

            pltpu.make_async_copy(tok_hbm.at[idx_v.at[g]], gbuf_b, gsem_b).wait()

            grow = jnp.full((16,), g, jnp.int32)
            for d in range(_D):
                cd = (lane + d) & 31
                c0 = cd >> 3
                c1 = cd & 7
                padd = plsc.load_gather(pos_v, [grow, cd])
                vs = [plsc.load_gather(gbuf_b, [rows[j], cd])
                      for j in range(8)]
                for j in range(8):
                    plsc.store_scatter(tbuf_b, [c0, c1, rows[j]], vs[j] + padd)

            pltpu.async_copy(tbuf_b, out_hbm.at[g, :, wid], ssem_b)

        def outer(i, carry):
            g = i * 2
            step(g, gbuf0, gsem0, ssem0, tbuf0, gbuf1, gsem1, ssem1, tbuf1)
            step(g + 1, gbuf1, gsem1, ssem1, tbuf1, gbuf0, gsem0, ssem0, tbuf0)
            return carry

        lax.fori_loop(0, _L // 2, outer, 0)

        pltpu.make_async_copy(tbuf1, out_hbm.at[_L - 1, :, wid], ssem1).wait()

    return emb_kernel, nw


def kernel(x, token_table, pos_table):
    emb, nw = _make_kernel()
    xt = jnp.transpose(x.astype(jnp.int32))
    out5 = emb(token_table, xt, pos_table)
    out = jnp.transpose(out5, (2, 4, 0, 1, 3))
    return out.reshape(_B, _L, _D)


# speedup vs baseline: 1.9061x; 1.3608x over previous
import functools

import jax
import jax.numpy as jnp
from jax import lax
from jax.experimental import pallas as pl
from jax.experimental.pallas import tpu as pltpu
from jax.experimental.pallas import tpu_sc as plsc

_B, _L, _D = 4096, 200, 32
_CH = 128


def _make_kernel():
    mesh = plsc.VectorSubcoreMesh(core_axis_name="c", subcore_axis_name="s")
    nc, ns = mesh.num_cores, mesh.num_subcores
    nw = nc * ns
    assert _B // _CH == nw

    @functools.partial(
        pl.kernel,
        out_type=jax.ShapeDtypeStruct((_L, _D // 8, nw, 8, _CH), jnp.float32),
        mesh=mesh,
        compiler_params=pltpu.CompilerParams(use_tc_tiling_on_sc=False,
                                             needs_layout_passes=False),
        scratch_types=[
            pltpu.VMEM((_L, _CH), jnp.int32),
            pltpu.VMEM((_L, _D), jnp.float32),
            pltpu.VMEM((_CH, _D), jnp.float32),
            pltpu.VMEM((_CH, _D), jnp.float32),
            pltpu.VMEM((_D // 8, 8, _CH + 1), jnp.float32),
            pltpu.VMEM((_D // 8, 8, _CH + 1), jnp.float32),
            pltpu.SemaphoreType.DMA,
            pltpu.SemaphoreType.DMA,
            pltpu.SemaphoreType.DMA,
            pltpu.SemaphoreType.DMA,
        ],
    )
    def emb_kernel(tok_hbm, xt_hbm, pos_hbm, out_hbm,
                   idx_v, pos_v, gbuf0, gbuf1, tbuf0, tbuf1,
                   gsem0, gsem1, ssem0, ssem1):
        wid = lax.axis_index("s") * nc + lax.axis_index("c")

        pltpu.sync_copy(xt_hbm.at[:, pl.ds(wid * _CH, _CH)], idx_v)
        pltpu.sync_copy(pos_hbm, pos_v)

        pltpu.async_copy(tok_hbm.at[idx_v.at[0]], gbuf0, gsem0)

        lane = lax.iota(jnp.int32, 16)
        zero = lane * 0
        lo0, lo1 = lane >> 3, lane & 7
        hi = lane + 16
        hi0, hi1 = hi >> 3, hi & 7

        def step(g, gbuf_b, gsem_b, ssem_b, tbuf_b, gbuf_n, gsem_n, ssem_n,
                 tbuf_n):
            @pl.when(g >= 1)
            def _():
                pltpu.make_async_copy(
                    tbuf_n.at[:, :, pl.ds(0, _CH)],
                    out_hbm.at[g - 1, :, wid], ssem_n).wait()

            @pl.when(g + 1 < _L)
            def _():
                pltpu.async_copy(tok_hbm.at[idx_v.at[g + 1]], gbuf_n, gsem_n)

            pltpu.make_async_copy(tok_hbm.at[idx_v.at[g]], gbuf_b, gsem_b).wait()

            pos_lo = pos_v[g, pl.ds(0, 16)]
            pos_hi = pos_v[g, pl.ds(16, 16)]
            for b0 in range(0, _CH, 8):
                vlo = [gbuf_b[b0 + k, pl.ds(0, 16)] + pos_lo for k in range(8)]
                vhi = [gbuf_b[b0 + k, pl.ds(16, 16)] + pos_hi for k in range(8)]
                for k in range(8):
                    cb = zero + (b0 + k)
                    plsc.store_scatter(tbuf_b, [lo0, lo1, cb], vlo[k])
                    plsc.store_scatter(tbuf_b, [hi0, hi1, cb], vhi[k])

            pltpu.async_copy(tbuf_b.at[:, :, pl.ds(0, _CH)],
                             out_hbm.at[g, :, wid], ssem_b)

        def outer(i, carry):
            g = i * 2
            step(g, gbuf0, gsem0, ssem0, tbuf0, gbuf1, gsem1, ssem1, tbuf1)
            step(g + 1, gbuf1, gsem1, ssem1, tbuf1, gbuf0, gsem0, ssem0, tbuf0)
            return carry

        lax.fori_loop(0, _L // 2, outer, 0)

        pltpu.make_async_copy(tbuf1.at[:, :, pl.ds(0, _CH)],
                              out_hbm.at[_L - 1, :, wid], ssem1).wait()

    return emb_kernel, nw


def kernel(x, token_table, pos_table):
    emb, nw = _make_kernel()
    xt = jnp.transpose(x.astype(jnp.int32))
    out5 = emb(token_table, xt, pos_table)
    out = jnp.transpose(out5, (2, 4, 0, 1, 3))
    return out.reshape(_B, _L, _D)
